# Initial kernel scaffold; baseline (speedup 1.0000x reference)
import jax
import jax.numpy as jnp
from jax.experimental import pallas as pl


def kernel(x, table, bias):
    raise NotImplementedError("write your pallas kernel here")


# speedup vs baseline: 1.4450x; 1.4450x over previous
import functools

import jax
import jax.numpy as jnp
from jax import lax
from jax.experimental import pallas as pl
from jax.experimental.pallas import tpu as pltpu
from jax.experimental.pallas import tpu_sc as plsc

BATCH = 16384
N_FIELDS = 26
FIELD_ROWS = 40000
NC = 2
NS = 16
NW = NC * NS
BPW = BATCH // NW
CHUNK = 128
NCHUNK = BPW // CHUNK
LANES = 16


def _sc_body(idx_hbm, table_hbm, bias_hbm, out_hbm, idx_v, rows_v, bias_v,
             out_v, sem):
    wid = lax.axis_index("s") * NC + lax.axis_index("c")
    pltpu.sync_copy(idx_hbm.at[wid], idx_v)
    pltpu.sync_copy(bias_hbm, bias_v)
    handles = []
    for f in range(N_FIELDS):
        for c in range(NCHUNK):
            handles.append(
                pltpu.async_copy(table_hbm.at[idx_v.at[f, c]],
                                 rows_v.at[f, c], sem))
    for h in handles:
        h.wait()

    bvec = bias_v[...]

    def body(k, carry):
        for c in range(NCHUNK):
            acc = bvec
            for f in range(N_FIELDS):
                acc = acc + rows_v[f, c, pl.ds(k * LANES, LANES)]
            out_v[pl.ds(c * CHUNK + k * LANES, LANES)] = acc
        return carry

    lax.fori_loop(0, CHUNK // LANES, body, 0)
    pltpu.sync_copy(out_v, out_hbm.at[pl.ds(wid * BPW, BPW)])


@jax.jit
def _embed_sum(idx_w, table_flat, bias16):
    mesh = plsc.VectorSubcoreMesh(core_axis_name="c", subcore_axis_name="s")
    call = functools.partial(
        pl.kernel,
        mesh=mesh,
        out_type=jax.ShapeDtypeStruct((BATCH,), jnp.float32),
        scratch_types=[
            pltpu.VMEM((N_FIELDS, NCHUNK, CHUNK), jnp.int32),
            pltpu.VMEM((N_FIELDS, NCHUNK, CHUNK), jnp.float32),
            pltpu.VMEM((LANES,), jnp.float32),
            pltpu.VMEM((BPW,), jnp.float32),
            pltpu.SemaphoreType.DMA,
        ],
    )(_sc_body)
    return call(idx_w, table_flat, bias16)


def kernel(x, table, bias):
    offsets = jnp.arange(N_FIELDS, dtype=jnp.int32) * FIELD_ROWS
    idx = x.astype(jnp.int32) + offsets[None, :]
    idx_w = (idx.T.reshape(N_FIELDS, NW, BPW)
             .transpose(1, 0, 2)
             .reshape(NW, N_FIELDS, NCHUNK, CHUNK))
    table_flat = table.reshape(-1)
    bias16 = jnp.broadcast_to(bias.astype(jnp.float32), (LANES,))
    out = _embed_sum(idx_w, table_flat, bias16)
    return out.reshape(BATCH, 1)
